# baseline (speedup 1.0000x reference)
import math

import jax
import jax.numpy as jnp
from jax import lax
from jax.experimental import pallas as pl
from jax.experimental.pallas import tpu as pltpu
from jax.experimental.pallas import tpu_sc as plsc

NC = 2
NS = 16
L = 16
NW = NC * NS

NSEG = 256
CHUNK = 256
BLK = 128
CROW = 384


def _sc_pool(h, batch, nblk_per_tile, interpret=False):
    N, H = h.shape
    HJ = H // L
    segs_per_w = NSEG // NW
    mesh = plsc.VectorSubcoreMesh(
        core_axis_name="c", subcore_axis_name="s", num_cores=NC, num_subcores=NS
    )
    cnt_pad = NSEG + L

    def body(h_hbm, batch_hbm, mean_hbm, var_hbm, max_hbm,
             counts_sp, idxbuf, onesbuf, rowbuf, cnt2d, endbuf, hbuf,
             meanbuf, varbuf, maxbuf, dsem, ssem):
        cid = lax.axis_index("c")
        sid = lax.axis_index("s")
        wid = cid * NS + sid

        zero = jnp.zeros((L,), jnp.float32)
        one = jnp.ones((L,), jnp.float32)
        for j in range(CROW // L):
            rowbuf[pl.ds(j * L, L)] = zero
        for j in range(BLK // L):
            onesbuf[pl.ds(j * L, L)] = one

        scope_a = jax.named_scope("phaseA")
        scope_a.__enter__()
        pltpu.sync_copy(rowbuf, counts_sp.at[pl.ds(sid * CROW, CROW)])

        ids0 = sid * nblk_per_tile * BLK
        full_blocks = N // BLK
        rem = N - full_blocks * BLK
        last_tile = full_blocks // nblk_per_tile
        fb_last = full_blocks - last_tile * nblk_per_tile

        @pl.when(sid >= last_tile)
        def _():
            nseg_v = jnp.broadcast_to(jnp.int32(NSEG), (L,))
            for r in range(fb_last, nblk_per_tile):
                for kk in range(BLK // L):
                    idxbuf[r, pl.ds(kk * L, L)] = nseg_v

        n_full = jnp.where(sid == last_tile, fb_last, nblk_per_tile)

        def idx_dma_body(j, carry):
            pltpu.async_copy(batch_hbm.at[pl.ds(ids0 + j * BLK, BLK)],
                             idxbuf.at[j], ssem)
            return carry

        lax.fori_loop(0, n_full, idx_dma_body, 0)

        if rem > 0:
            @pl.when(sid == last_tile)
            def _():
                pltpu.async_copy(
                    batch_hbm.at[pl.ds(full_blocks * BLK, rem)],
                    idxbuf.at[fb_last, pl.ds(0, rem)], ssem)
                pltpu.make_async_copy(
                    batch_hbm.at[pl.ds(full_blocks * BLK, rem)],
                    idxbuf.at[fb_last, pl.ds(0, rem)], ssem).wait()

        def idx_drain_body(j, carry):
            pltpu.make_async_copy(batch_hbm.at[pl.ds(ids0 + j * BLK, BLK)],
                                  idxbuf.at[j], ssem).wait()
            return carry

        lax.fori_loop(0, n_full, idx_drain_body, 0)

        boff = jnp.broadcast_to(sid * CROW, (L,)).astype(jnp.int32)

        def bias_body(j, carry):
            for kk in range(BLK // L):
                sl = pl.ds(kk * L, L)
                idxbuf[j, sl] = idxbuf[j, sl] + boff
            return carry

        lax.fori_loop(0, nblk_per_tile, bias_body, 0)

        def scatter_body(j, carry):
            pltpu.async_copy(onesbuf, counts_sp.at[idxbuf.at[j]], ssem,
                             add=True)
            return carry

        lax.fori_loop(0, nblk_per_tile, scatter_body, 0)

        def drain_body(j, carry):
            pltpu.make_async_copy(onesbuf, counts_sp.at[idxbuf.at[j]],
                                  ssem).wait()
            return carry

        lax.fori_loop(0, nblk_per_tile, drain_body, 0)
        pltpu.sync_copy(counts_sp.at[pl.ds(sid * CROW, CROW)], rowbuf)
        plsc.subcore_barrier()
        scope_a.__exit__(None, None, None)
        scope_b = jax.named_scope("phaseB")
        scope_b.__enter__()

        pltpu.sync_copy(counts_sp, cnt2d)
        run = jnp.int32(0)
        for j in range(NSEG // L):
            acc = jnp.zeros((L,), jnp.float32)
            for r in range(NS):
                acc = acc + cnt2d[pl.ds(r * CROW + j * L, L)]
            v = acc.astype(jnp.int32)
            endbuf[pl.ds(j * L, L)] = plsc.cumsum(v) + run
            run = run + jnp.sum(v)
        endbuf[pl.ds(NSEG, L)] = jnp.broadcast_to(run, (L,))

        scope_b.__exit__(None, None, None)
        scope_c = jax.named_scope("phaseC")
        scope_c.__enter__()
        b0 = wid * segs_per_w
        S = jnp.where(b0 == 0, 0, endbuf[pl.ds(jnp.maximum(b0 - 1, 0), L)][0])
        E = endbuf[pl.ds(b0 + segs_per_w - 1, L)][0]
        S8 = S & ~7
        nch = (E - S8 + CHUNK - 1) >> 8

        def win(kc):
            return pl.multiple_of(
                jnp.minimum(S8 + kc * CHUNK, N - CHUNK) & ~7, 8)

        def start_dma(kc, p):
            pltpu.async_copy(h_hbm.at[pl.ds(win(kc), CHUNK)], hbuf.at[p],
                             dsem.at[p])

        @pl.when(nch > 0)
        def _():
            start_dma(0, 0)

        @pl.when(nch > 1)
        def _():
            start_dma(1, 1)

        zacc = tuple(jnp.zeros((L,), jnp.float32) for _ in range(2 * HJ))
        macc = tuple(jnp.full((L,), -jnp.inf, jnp.float32) for _ in range(HJ))

        def add_row(p, i, acc):
            sums = acc[:HJ]
            sqs = acc[HJ:2 * HJ]
            mxs = acc[2 * HJ:]
            out = []
            xs = [hbuf[p, i, pl.ds(j * L, L)] for j in range(HJ)]
            out.extend(sums[j] + xs[j] for j in range(HJ))
            out.extend(sqs[j] + xs[j] * xs[j] for j in range(HJ))
            out.extend(jnp.maximum(mxs[j], xs[j]) for j in range(HJ))
            return tuple(out)

        def accum_rows(p, lo, hi, acc):
            n = hi - lo

            def body4(i4, a):
                base = lo + i4 * 4
                for u in range(4):
                    a = add_row(p, base + u, a)
                return a

            acc = lax.fori_loop(0, n >> 2, body4, acc)
            return lax.fori_loop(lo + (n & ~3), hi, add_row_p(p), acc)

        def add_row_p(p):
            return lambda i, a: add_row(p, i, a)

        def finalize(bcur, seg_start, seg_end, acc):
            cnt = seg_end - seg_start
            cnt_v = jnp.broadcast_to(cnt, (L,)).astype(jnp.float32)
            rcv = 1.0 / jnp.maximum(cnt_v, 1.0)
            row = bcur - b0
            for j in range(HJ):
                m = acc[j] * rcv
                v = jnp.maximum(acc[HJ + j] * rcv - m * m, 0.0)
                mx = jnp.where(cnt > 0, acc[2 * HJ + j], 0.0)
                meanbuf[row, pl.ds(j * L, L)] = m
                varbuf[row, pl.ds(j * L, L)] = v
                maxbuf[row, pl.ds(j * L, L)] = mx

        def drain_segments(p, wb, chi_g, st):
            def cond(st):
                return (st[0] < b0 + segs_per_w) & (st[1] <= chi_g)

            def body(st):
                bcur, e_cur, seg_start, rp = st[:4]
                acc = st[4:]
                acc = accum_rows(p, rp - wb, e_cur - wb, acc)
                finalize(bcur, seg_start, e_cur, acc)
                bnew = bcur + 1
                enew = endbuf[pl.ds(jnp.minimum(bnew, NSEG - 1), L)][0]
                return (bnew, enew, e_cur, e_cur) + zacc + macc

            return lax.while_loop(cond, body, st)

        def chunk_body(kc, st):
            p = kc & 1
            wb = win(kc)
            chi_g = jnp.minimum(E, wb + CHUNK)
            pltpu.make_async_copy(h_hbm.at[pl.ds(wb, CHUNK)], hbuf.at[p],
                                  dsem.at[p]).wait()
            st = drain_segments(p, wb, chi_g, st)
            bcur, e_cur, seg_start, rp = st[:4]
            acc = st[4:]
            acc = accum_rows(p, rp - wb, chi_g - wb, acc)

            @pl.when(kc + 2 < nch)
            def _():
                start_dma(kc + 2, p)

            return (bcur, e_cur, seg_start, chi_g) + acc

        e0 = endbuf[pl.ds(b0, L)][0]
        st = (b0, e0, S, S) + zacc + macc
        st = lax.fori_loop(0, nch, chunk_body, st)
        st = drain_segments(0, 0, E, st)

        scope_c.__exit__(None, None, None)
        base_row = wid * segs_per_w
        pltpu.sync_copy(meanbuf, mean_hbm.at[pl.ds(base_row, segs_per_w)])
        pltpu.sync_copy(varbuf, var_hbm.at[pl.ds(base_row, segs_per_w)])
        pltpu.sync_copy(maxbuf, max_hbm.at[pl.ds(base_row, segs_per_w)])

    f32 = jnp.float32
    out = jax.ShapeDtypeStruct((NSEG, H), f32)
    call = pl.kernel(
        body,
        out_type=(out, out, out),
        mesh=mesh,
        scratch_types=[
            pltpu.VMEM_SHARED((NS * CROW,), f32),
            pltpu.VMEM((nblk_per_tile, BLK), jnp.int32),
            pltpu.VMEM((BLK,), f32),
            pltpu.VMEM((CROW,), f32),
            pltpu.VMEM((NS * CROW,), f32),
            pltpu.VMEM((NSEG + L,), jnp.int32),
            pltpu.VMEM((2, CHUNK, H), f32),
            pltpu.VMEM((NSEG // NW, H), f32),
            pltpu.VMEM((NSEG // NW, H), f32),
            pltpu.VMEM((NSEG // NW, H), f32),
            pltpu.SemaphoreType.DMA((2,)),
            pltpu.SemaphoreType.DMA,
        ],
        compiler_params=pltpu.CompilerParams(needs_layout_passes=False),
        interpret=interpret,
    )
    return call(h, batch)


def _mlp_body(mean_ref, var_ref, max_ref, w1_ref, b1_ref, w2_ref, b2_ref,
              out_ref):
    std = jnp.sqrt(var_ref[...] + 1e-8)
    f32 = jnp.float32
    H = mean_ref.shape[1]
    hid = (jnp.dot(mean_ref[...], w1_ref[:H], preferred_element_type=f32)
           + jnp.dot(max_ref[...], w1_ref[H:2 * H], preferred_element_type=f32)
           + jnp.dot(std, w1_ref[2 * H:], preferred_element_type=f32))
    hid = jnp.maximum(hid + b1_ref[...], 0.0)
    z = jnp.dot(hid, w2_ref[...], preferred_element_type=jnp.float32)
    out_ref[...] = jnp.tanh(z + b2_ref[...]) * math.pi


def kernel(h, batch, W1, b1, W2, b2):
    N, H = h.shape
    nblocks = -(-N // BLK)
    nblk_per_tile = -(-nblocks // NS)

    g_mean, g_var, g_max = _sc_pool(h, batch.astype(jnp.int32), nblk_per_tile)

    z = pl.pallas_call(
        _mlp_body,
        out_shape=jax.ShapeDtypeStruct((NSEG, W2.shape[1]), jnp.float32),
    )(g_mean, g_var, g_max, W1, b1.reshape(1, -1), W2, b2.reshape(1, -1))
    return z


# speedup vs baseline: 1.0751x; 1.0751x over previous
import math

import jax
import jax.numpy as jnp
from jax import lax
from jax.experimental import pallas as pl
from jax.experimental.pallas import tpu as pltpu
from jax.experimental.pallas import tpu_sc as plsc

NC = 2
NS = 16
L = 16
NW = NC * NS

NSEG = 256
CHUNK = 256
BLK = 128
SROW = 512
S2LEN = ((NSEG + L) // L + 2) * L


def _sc_pool(h, batch, nblk_per_tile, interpret=False):
    N, H = h.shape
    HJ = H // L
    segs_per_w = NSEG // NW
    mesh = plsc.VectorSubcoreMesh(
        core_axis_name="c", subcore_axis_name="s", num_cores=NC, num_subcores=NS
    )

    def body(h_hbm, batch_hbm, mean_hbm, var_hbm, max_hbm,
             starts_sp, idxflat, starts_local, st2d, s2buf, hbuf,
             meanbuf, varbuf, maxbuf, dsem):
        cid = lax.axis_index("c")
        sid = lax.axis_index("s")
        wid = cid * NS + sid

        scope_a = jax.named_scope("phaseA")
        scope_a.__enter__()
        ids_per_tile = nblk_per_tile * BLK
        ids0 = sid * ids_per_tile
        last_tile = (N // BLK) // nblk_per_tile
        valid_last = N - last_tile * ids_per_tile
        iota = lax.iota(jnp.int32, L)
        bigN = jnp.broadcast_to(jnp.int32(N), (L,))

        for g in range(SROW // L):
            starts_local[pl.ds(g * L, L)] = bigN

        idxflat[pl.ds(0, L)] = jnp.where(iota == L - 1, -1, 0)

        @pl.when(sid > 0)
        def _():
            pltpu.sync_copy(batch_hbm.at[pl.ds(ids0 - 8, 8)],
                            idxflat.at[pl.ds(8, 8)])

        @pl.when(sid < last_tile)
        def _():
            pltpu.sync_copy(batch_hbm.at[pl.ds(ids0, ids_per_tile)],
                            idxflat.at[pl.ds(L, ids_per_tile)])

        @pl.when(sid == last_tile)
        def _():
            nseg_v = jnp.broadcast_to(jnp.int32(NSEG), (L,))
            for g in range(valid_last // L, ids_per_tile // L):
                idxflat[pl.ds(L + g * L, L)] = nseg_v
            pltpu.sync_copy(batch_hbm.at[pl.ds(ids0, valid_last // 8 * 8)],
                            idxflat.at[pl.ds(L, valid_last // 8 * 8)])

        if valid_last % 8 != 0:
            @pl.when(sid == last_tile)
            def _():
                off = valid_last // 8 * 8
                pltpu.sync_copy(
                    batch_hbm.at[pl.ds(ids0 + off, valid_last - off)],
                    idxflat.at[pl.ds(L + off, valid_last - off)])

        def group4(g4, carry):
            for u in range(4):
                g = g4 * 4 + u
                cur = idxflat[pl.ds(L + g * L, L)]
                prev = idxflat[pl.ds(L - 1 + g * L, L)]
                mask = cur != prev
                pos = (jnp.broadcast_to(ids0 + g * L, (L,)).astype(jnp.int32)
                       + iota)
                plsc.store_scatter(starts_local, [cur], pos, mask=mask)
            return carry

        lax.fori_loop(0, ids_per_tile // L // 4, group4, 0)

        pltpu.sync_copy(starts_local, starts_sp.at[pl.ds(sid * SROW, SROW)])
        pltpu.sync_copy(starts_sp.at[pl.ds(sid * SROW, SROW)], starts_local)
        plsc.subcore_barrier()
        scope_a.__exit__(None, None, None)
        scope_b = jax.named_scope("phaseB")
        scope_b.__enter__()

        pltpu.sync_copy(starts_sp, st2d)
        NGC = (NSEG + L) // L + 1
        s2buf[pl.ds(NGC * L, L)] = bigN
        carry = jnp.int32(N)
        for g in range(NGC - 1, -1, -1):
            v = bigN
            for r in range(NS):
                v = jnp.minimum(v, st2d[pl.ds(r * SROW + g * L, L)])
            sm = -lax.rev(plsc.cummax(lax.rev(-v, (0,))), (0,))
            s2buf[pl.ds(g * L, L)] = jnp.minimum(sm, carry)
            carry = jnp.minimum(carry, jnp.min(v))

        scope_b.__exit__(None, None, None)
        scope_c = jax.named_scope("phaseC")
        scope_c.__enter__()
        b0 = wid * segs_per_w
        S = s2buf[pl.ds(b0, L)][0]
        E = s2buf[pl.ds(b0 + segs_per_w, L)][0]
        S8 = S & ~7
        nch = (E - S8 + CHUNK - 1) >> 8

        def win(kc):
            return pl.multiple_of(
                jnp.minimum(S8 + kc * CHUNK, N - CHUNK) & ~7, 8)

        def start_dma(kc, p):
            pltpu.async_copy(h_hbm.at[pl.ds(win(kc), CHUNK)], hbuf.at[p],
                             dsem.at[p])

        @pl.when(nch > 0)
        def _():
            start_dma(0, 0)

        @pl.when(nch > 1)
        def _():
            start_dma(1, 1)

        zacc = tuple(jnp.zeros((L,), jnp.float32) for _ in range(2 * HJ))
        macc = tuple(jnp.full((L,), -jnp.inf, jnp.float32) for _ in range(HJ))

        def add_row(p, i, acc):
            sums = acc[:HJ]
            sqs = acc[HJ:2 * HJ]
            mxs = acc[2 * HJ:]
            out = []
            xs = [hbuf[p, i, pl.ds(j * L, L)] for j in range(HJ)]
            out.extend(sums[j] + xs[j] for j in range(HJ))
            out.extend(sqs[j] + xs[j] * xs[j] for j in range(HJ))
            out.extend(jnp.maximum(mxs[j], xs[j]) for j in range(HJ))
            return tuple(out)

        def accum_rows(p, lo, hi, acc):
            n = hi - lo

            def body4(i4, a):
                base = lo + i4 * 4
                for u in range(4):
                    a = add_row(p, base + u, a)
                return a

            acc = lax.fori_loop(0, n >> 2, body4, acc)
            return lax.fori_loop(lo + (n & ~3), hi, add_row_p(p), acc)

        def add_row_p(p):
            return lambda i, a: add_row(p, i, a)

        def finalize(bcur, seg_start, seg_end, acc):
            cnt = seg_end - seg_start
            cnt_v = jnp.broadcast_to(cnt, (L,)).astype(jnp.float32)
            rcv = 1.0 / jnp.maximum(cnt_v, 1.0)
            row = bcur - b0
            for j in range(HJ):
                m = acc[j] * rcv
                v = jnp.maximum(acc[HJ + j] * rcv - m * m, 0.0)
                mx = jnp.where(cnt > 0, acc[2 * HJ + j], 0.0)
                meanbuf[row, pl.ds(j * L, L)] = m
                varbuf[row, pl.ds(j * L, L)] = v
                maxbuf[row, pl.ds(j * L, L)] = mx

        def drain_segments(p, wb, chi_g, st):
            def cond(st):
                return (st[0] < b0 + segs_per_w) & (st[1] <= chi_g)

            def body(st):
                bcur, e_cur, seg_start, rp = st[:4]
                acc = st[4:]
                acc = accum_rows(p, rp - wb, e_cur - wb, acc)
                finalize(bcur, seg_start, e_cur, acc)
                bnew = bcur + 1
                enew = s2buf[pl.ds(jnp.minimum(bnew + 1, NSEG), L)][0]
                return (bnew, enew, e_cur, e_cur) + zacc + macc

            return lax.while_loop(cond, body, st)

        def chunk_body(kc, st):
            p = kc & 1
            wb = win(kc)
            chi_g = jnp.minimum(E, wb + CHUNK)
            pltpu.make_async_copy(h_hbm.at[pl.ds(wb, CHUNK)], hbuf.at[p],
                                  dsem.at[p]).wait()
            st = drain_segments(p, wb, chi_g, st)
            bcur, e_cur, seg_start, rp = st[:4]
            acc = st[4:]
            acc = accum_rows(p, rp - wb, chi_g - wb, acc)

            @pl.when(kc + 2 < nch)
            def _():
                start_dma(kc + 2, p)

            return (bcur, e_cur, seg_start, chi_g) + acc

        e0 = s2buf[pl.ds(b0 + 1, L)][0]
        st = (b0, e0, S, S) + zacc + macc
        st = lax.fori_loop(0, nch, chunk_body, st)
        st = drain_segments(0, 0, E, st)

        scope_c.__exit__(None, None, None)
        base_row = wid * segs_per_w
        pltpu.sync_copy(meanbuf, mean_hbm.at[pl.ds(base_row, segs_per_w)])
        pltpu.sync_copy(varbuf, var_hbm.at[pl.ds(base_row, segs_per_w)])
        pltpu.sync_copy(maxbuf, max_hbm.at[pl.ds(base_row, segs_per_w)])

    f32 = jnp.float32
    out = jax.ShapeDtypeStruct((NSEG, H), f32)
    call = pl.kernel(
        body,
        out_type=(out, out, out),
        mesh=mesh,
        scratch_types=[
            pltpu.VMEM_SHARED((NS * SROW,), jnp.int32),
            pltpu.VMEM((L + nblk_per_tile * BLK,), jnp.int32),
            pltpu.VMEM((SROW,), jnp.int32),
            pltpu.VMEM((NS * SROW,), jnp.int32),
            pltpu.VMEM((S2LEN,), jnp.int32),
            pltpu.VMEM((2, CHUNK, H), f32),
            pltpu.VMEM((NSEG // NW, H), f32),
            pltpu.VMEM((NSEG // NW, H), f32),
            pltpu.VMEM((NSEG // NW, H), f32),
            pltpu.SemaphoreType.DMA((2,)),
        ],
        compiler_params=pltpu.CompilerParams(needs_layout_passes=False),
        interpret=interpret,
    )
    return call(h, batch)


def _mlp_body(mean_ref, var_ref, max_ref, w1_ref, b1_ref, w2_ref, b2_ref,
              out_ref):
    std = jnp.sqrt(var_ref[...] + 1e-8)
    f32 = jnp.float32
    H = mean_ref.shape[1]
    hid = (jnp.dot(mean_ref[...], w1_ref[:H], preferred_element_type=f32)
           + jnp.dot(max_ref[...], w1_ref[H:2 * H], preferred_element_type=f32)
           + jnp.dot(std, w1_ref[2 * H:], preferred_element_type=f32))
    hid = jnp.maximum(hid + b1_ref[...], 0.0)
    z = jnp.dot(hid, w2_ref[...], preferred_element_type=jnp.float32)
    out_ref[...] = jnp.tanh(z + b2_ref[...]) * math.pi


def kernel(h, batch, W1, b1, W2, b2):
    N, H = h.shape
    nblocks = -(-N // BLK)
    nblk_per_tile = -(-nblocks // NS)

    g_mean, g_var, g_max = _sc_pool(h, batch.astype(jnp.int32), nblk_per_tile)

    z = pl.pallas_call(
        _mlp_body,
        out_shape=jax.ShapeDtypeStruct((NSEG, W2.shape[1]), jnp.float32),
    )(g_mean, g_var, g_max, W1, b1.reshape(1, -1), W2, b2.reshape(1, -1))
    return z
